# Initial kernel scaffold; baseline (speedup 1.0000x reference)
import jax
import jax.numpy as jnp
from jax.experimental import pallas as pl


def kernel(z, edge_index):
    raise NotImplementedError("write your pallas kernel here")


# speedup vs baseline: 1.4652x; 1.4652x over previous
import functools

import jax
import jax.numpy as jnp
from jax import lax
from jax.experimental import pallas as pl
from jax.experimental.pallas import tpu as pltpu
from jax.experimental.pallas import tpu_sc as plsc

N = 10000
D = 128
E = 320000
NC, NS, L = 2, 16, 16
NW = NC * NS
EPW = E // NW
C = 80
NCH = EPW // C
G = C // L
EPS = 1e-6


def _normalize_body(z_ref, out_ref):
    z = z_ref[...]
    n = jnp.sqrt(jnp.sum(z * z, axis=1, keepdims=True))
    out_ref[...] = z / n


def _normalize(z):
    blk = N // 10
    return pl.pallas_call(
        _normalize_body,
        out_shape=jax.ShapeDtypeStruct((N, D), jnp.float32),
        grid=(10,),
        in_specs=[pl.BlockSpec((blk, D), lambda i: (i, 0))],
        out_specs=pl.BlockSpec((blk, D), lambda i: (i, 0)),
    )(z)


def _rsqrt_newton(x):
    xi = plsc.bitcast(x, jnp.int32)
    yi = jnp.int32(0x5F3759DF) - (xi >> 1)
    y = plsc.bitcast(yi, jnp.float32)
    for _ in range(3):
        y = y * (1.5 - 0.5 * x * y * y)
    return y


def _edge_body(zn_hbm, src_hbm, dst_hbm, out_hbm, si_v, di_v, a_v, b_v, o_v, sem):
    wid = lax.axis_index("s") * NC + lax.axis_index("c")
    base = pl.multiple_of(wid * EPW, 8)
    pltpu.sync_copy(src_hbm.at[pl.ds(base, EPW)], si_v)
    pltpu.sync_copy(dst_hbm.at[pl.ds(base, EPW)], di_v)

    row16 = lax.iota(jnp.int32, 16)

    def chunk(j, carry):
        off = pl.multiple_of(j * C, 8)
        cp_a = pltpu.async_copy(zn_hbm.at[si_v.at[pl.ds(off, C)]], a_v, sem)
        cp_b = pltpu.async_copy(zn_hbm.at[di_v.at[pl.ds(off, C)]], b_v, sem)
        cp_a.wait()
        cp_b.wait()
        for g in range(G):
            rows = row16 + (g * L)

            def kbody(kk, acc):
                for u in range(16):
                    k = kk * 16 + u
                    col = jnp.full((16,), 0, jnp.int32) + k
                    va = plsc.load_gather(a_v, [rows, col])
                    vb = plsc.load_gather(b_v, [rows, col])
                    t = va - vb + EPS
                    acc = acc + t * t
                return acc

            x = lax.fori_loop(0, 8, kbody, jnp.zeros((16,), jnp.float32))
            d = x * _rsqrt_newton(x)
            o = 1.0 / (1.0 + jnp.exp(d - 1.0))
            o_v[pl.ds(pl.multiple_of(off + g * L, 8), L)] = o
        return carry

    lax.fori_loop(0, NCH, chunk, 0)
    pltpu.sync_copy(o_v, out_hbm.at[pl.ds(base, EPW)])


_edge_kernel = functools.partial(
    pl.kernel,
    out_type=jax.ShapeDtypeStruct((E,), jnp.float32),
    mesh=plsc.VectorSubcoreMesh(
        core_axis_name="c", subcore_axis_name="s", num_cores=NC, num_subcores=NS
    ),
    scratch_types=[
        pltpu.VMEM((EPW,), jnp.int32),
        pltpu.VMEM((EPW,), jnp.int32),
        pltpu.VMEM((C, D), jnp.float32),
        pltpu.VMEM((C, D), jnp.float32),
        pltpu.VMEM((EPW,), jnp.float32),
        pltpu.SemaphoreType.DMA,
    ],
    compiler_params=pltpu.CompilerParams(needs_layout_passes=False),
)(_edge_body)


@jax.jit
def kernel(z, edge_index):
    zn = _normalize(z)
    return _edge_kernel(zn, edge_index[0], edge_index[1])


# speedup vs baseline: 1.8467x; 1.2604x over previous
import functools

import jax
import jax.numpy as jnp
from jax import lax
from jax.experimental import pallas as pl
from jax.experimental.pallas import tpu as pltpu
from jax.experimental.pallas import tpu_sc as plsc

N = 10000
D = 128
E = 320000
NC, NS, L = 2, 16, 16
NW = NC * NS
EPW = E // NW
C = 80
NCH = EPW // C
G = C // L
NBUF = 5
NO = NCH // NBUF
EPS = 1e-6


def _normalize_body(z_ref, out_ref):
    z = z_ref[...]
    n = jnp.sqrt(jnp.sum(z * z, axis=1, keepdims=True))
    out_ref[...] = z / n


def _normalize(z):
    blk = N // 10
    return pl.pallas_call(
        _normalize_body,
        out_shape=jax.ShapeDtypeStruct((N, D), jnp.float32),
        grid=(10,),
        in_specs=[pl.BlockSpec((blk, D), lambda i: (i, 0))],
        out_specs=pl.BlockSpec((blk, D), lambda i: (i, 0)),
    )(z)


def _rsqrt_newton(x):
    xi = plsc.bitcast(x, jnp.int32)
    yi = jnp.int32(0x5F3759DF) - (xi >> 1)
    y = plsc.bitcast(yi, jnp.float32)
    for _ in range(3):
        y = y * (1.5 - 0.5 * x * y * y)
    return y


def _edge_body(zn_hbm, src_hbm, dst_hbm, out_hbm, si_v, di_v, a_bufs, b_bufs,
               o_v, sems):
    wid = lax.axis_index("s") * NC + lax.axis_index("c")
    base = pl.multiple_of(wid * EPW, 8)
    pltpu.sync_copy(src_hbm.at[pl.ds(base, EPW)], si_v)
    pltpu.sync_copy(dst_hbm.at[pl.ds(base, EPW)], di_v)

    row16 = lax.iota(jnp.int32, 16)

    def fire(j, b):
        off = pl.multiple_of(j * C, 8)
        pltpu.async_copy(zn_hbm.at[si_v.at[pl.ds(off, C)]], a_bufs[b], sems[b])
        pltpu.async_copy(zn_hbm.at[di_v.at[pl.ds(off, C)]], b_bufs[b], sems[b])

    def drain(b):
        pltpu.make_async_copy(
            zn_hbm.at[si_v.at[pl.ds(0, C)]], a_bufs[b], sems[b]).wait()
        pltpu.make_async_copy(
            zn_hbm.at[di_v.at[pl.ds(0, C)]], b_bufs[b], sems[b]).wait()

    def compute(b):
        a_v, b_v = a_bufs[b], b_bufs[b]

        def gbody(g, carry):
            rows = row16 + g * L

            def kbody(kk, accs):
                a0, a1, a2, a3 = accs
                base_k = kk * 16
                for u in range(16):
                    col = jnp.full((16,), 0, jnp.int32) + (base_k + u)
                    va = plsc.load_gather(a_v, [rows, col])
                    vb = plsc.load_gather(b_v, [rows, col])
                    t = va - vb + EPS
                    p = t * t
                    if u % 4 == 0:
                        a0 = a0 + p
                    elif u % 4 == 1:
                        a1 = a1 + p
                    elif u % 4 == 2:
                        a2 = a2 + p
                    else:
                        a3 = a3 + p
                return a0, a1, a2, a3

            z4 = jnp.zeros((16,), jnp.float32)
            a0, a1, a2, a3 = lax.fori_loop(0, 8, kbody, (z4, z4, z4, z4))
            x = (a0 + a1) + (a2 + a3)
            d = x * _rsqrt_newton(x)
            o = 1.0 / (1.0 + jnp.exp(d - 1.0))
            o_v[pl.ds(pl.multiple_of(b * C + g * L, 8), L)] = o
            return carry

        lax.fori_loop(0, G, gbody, 0)

    for b in range(NBUF):
        fire(b, b)

    def outer(t, carry):
        for b in range(NBUF):
            j = t * NBUF + b
            drain(b)
            compute(b)

            @pl.when(j + NBUF < NCH)
            def _():
                fire(j + NBUF, b)

        dst = out_hbm.at[pl.ds(pl.multiple_of(base + t * (NBUF * C), 8),
                               NBUF * C)]
        pltpu.sync_copy(o_v, dst)
        return carry

    lax.fori_loop(0, NO, outer, 0)


_edge_kernel = functools.partial(
    pl.kernel,
    out_type=jax.ShapeDtypeStruct((E,), jnp.float32),
    mesh=plsc.VectorSubcoreMesh(
        core_axis_name="c", subcore_axis_name="s", num_cores=NC, num_subcores=NS
    ),
    scratch_types=[
        pltpu.VMEM((EPW,), jnp.int32),
        pltpu.VMEM((EPW,), jnp.int32),
        [pltpu.VMEM((C, D), jnp.float32) for _ in range(NBUF)],
        [pltpu.VMEM((C, D), jnp.float32) for _ in range(NBUF)],
        pltpu.VMEM((NBUF * C,), jnp.float32),
        [pltpu.SemaphoreType.DMA for _ in range(NBUF)],
    ],
    compiler_params=pltpu.CompilerParams(needs_layout_passes=False),
)(_edge_body)


@jax.jit
def kernel(z, edge_index):
    zn = _normalize(z)
    return _edge_kernel(zn, edge_index[0], edge_index[1])


# speedup vs baseline: 11.8562x; 6.4202x over previous
import functools

import jax
import jax.numpy as jnp
from jax import lax
from jax.experimental import pallas as pl
from jax.experimental.pallas import tpu as pltpu
from jax.experimental.pallas import tpu_sc as plsc

N = 10000
D = 128
E = 320000
NC, NS, L = 2, 16, 16
NW = NC * NS
EPW = E // NW
C = 80
NCH = EPW // C
G = C // L
NBUF = 5
NO = NCH // NBUF
EPS = 1e-6


def _normalize_body(z_ref, out_ref):
    z = z_ref[...]
    n = jnp.sqrt(jnp.sum(z * z, axis=1, keepdims=True))
    out_ref[...] = z / n


def _normalize(z):
    blk = N // 10
    return pl.pallas_call(
        _normalize_body,
        out_shape=jax.ShapeDtypeStruct((N, D), jnp.float32),
        grid=(10,),
        in_specs=[pl.BlockSpec((blk, D), lambda i: (i, 0))],
        out_specs=pl.BlockSpec((blk, D), lambda i: (i, 0)),
    )(z)


def _rsqrt_newton(x):
    xi = plsc.bitcast(x, jnp.int32)
    yi = jnp.int32(0x5F3759DF) - (xi >> 1)
    y = plsc.bitcast(yi, jnp.float32)
    for _ in range(3):
        y = y * (1.5 - 0.5 * x * y * y)
    return y


def _edge_body(zn_hbm, src_hbm, dst_hbm, out_hbm, si_v, di_v, a_bufs, b_bufs,
               o_v, sems):
    wid = lax.axis_index("s") * NC + lax.axis_index("c")
    base = pl.multiple_of(wid * EPW, 8)
    pltpu.sync_copy(src_hbm.at[pl.ds(base, EPW)], si_v)
    pltpu.sync_copy(dst_hbm.at[pl.ds(base, EPW)], di_v)

    row16 = lax.iota(jnp.int32, 16)

    def fire(j, b):
        off = pl.multiple_of(j * C, 8)
        pltpu.async_copy(zn_hbm.at[si_v.at[pl.ds(off, C)]], a_bufs[b], sems[b])
        pltpu.async_copy(zn_hbm.at[di_v.at[pl.ds(off, C)]], b_bufs[b], sems[b])

    def drain(b):
        pltpu.make_async_copy(
            zn_hbm.at[si_v.at[pl.ds(0, C)]], a_bufs[b], sems[b]).wait()
        pltpu.make_async_copy(
            zn_hbm.at[di_v.at[pl.ds(0, C)]], b_bufs[b], sems[b]).wait()

    def compute(b):
        a_v, b_v = a_bufs[b], b_bufs[b]

        def gbody(g, carry):
            def quad(qq, x):
                for u4 in range(4):
                    u = qq * 4 + u4
                    e = g * L + u
                    acc = None
                    for kk in range(8):
                        va = a_v[e, pl.ds(kk * L, L)]
                        vb = b_v[e, pl.ds(kk * L, L)]
                        t = va - vb + EPS
                        p = t * t
                        acc = p if acc is None else acc + p
                    x = jnp.where(row16 == u, jnp.sum(acc), x)
                return x

            x = lax.fori_loop(0, 4, quad, jnp.zeros((16,), jnp.float32))
            d = x * _rsqrt_newton(x)
            o = 1.0 / (1.0 + jnp.exp(d - 1.0))
            o_v[pl.ds(b * C + g * L, L)] = o
            return carry

        lax.fori_loop(0, G, gbody, 0)

    for b in range(NBUF):
        fire(b, b)

    def outer(t, carry):
        for b in range(NBUF):
            j = t * NBUF + b
            drain(b)
            compute(b)

            @pl.when(j + NBUF < NCH)
            def _():
                fire(j + NBUF, b)

        dst = out_hbm.at[pl.ds(pl.multiple_of(base + t * (NBUF * C), 8),
                               NBUF * C)]
        pltpu.sync_copy(o_v, dst)
        return carry

    lax.fori_loop(0, NO, outer, 0)


_edge_kernel = functools.partial(
    pl.kernel,
    out_type=jax.ShapeDtypeStruct((E,), jnp.float32),
    mesh=plsc.VectorSubcoreMesh(
        core_axis_name="c", subcore_axis_name="s", num_cores=NC, num_subcores=NS
    ),
    scratch_types=[
        pltpu.VMEM((EPW,), jnp.int32),
        pltpu.VMEM((EPW,), jnp.int32),
        [pltpu.VMEM((C, D), jnp.float32) for _ in range(NBUF)],
        [pltpu.VMEM((C, D), jnp.float32) for _ in range(NBUF)],
        pltpu.VMEM((NBUF * C,), jnp.float32),
        [pltpu.SemaphoreType.DMA for _ in range(NBUF)],
    ],
    compiler_params=pltpu.CompilerParams(needs_layout_passes=False),
)(_edge_body)


@jax.jit
def kernel(z, edge_index):
    zn = _normalize(z)
    return _edge_kernel(zn, edge_index[0], edge_index[1])
